# baseline (speedup 1.0000x reference)
import jax
import jax.numpy as jnp
from jax.experimental import pallas as pl
from jax.experimental.pallas import tpu as pltpu

DEPTH = 2
DIM = 64
NTOK = 21
TDIM = 16
MDIM = 16
N = 256
TI = 32
NI = N // TI

_SELU_ALPHA = 1.6732632423543772
_SELU_SCALE = 1.0507009873554805
_HI = jax.lax.Precision.HIGHEST


def _silu(x):
    return x * jax.nn.sigmoid(x)


def _selu(x):
    return _SELU_SCALE * jnp.where(x > 0, x, _SELU_ALPHA * (jnp.exp(x) - 1.0))


def _dot(a, b):
    return jnp.dot(a, b, preferred_element_type=jnp.float32, precision=_HI)


def _dotg(a, b, dims):
    return jax.lax.dot_general(a, b, (dims, ((), ())),
                               preferred_element_type=jnp.float32,
                               precision=_HI)


def _dotd(a, b):
    return jnp.dot(a, b, preferred_element_type=jnp.float32)


def _pair_tile(x_t, x_full, c_full, ci_t,
               w1_ref, b1_ref, w2_ref, b2_ref,
               c1_ref, bc1_ref, c2_ref, bc2_ref, lng_ref, lnb_ref,
               n1_ref, bn1_ref, n2_ref, bn2_ref):
    rel = ci_t[:, None, :] - c_full[None, :, :]
    d_t = jnp.sum(rel * rel, axis=-1)

    e = jnp.concatenate([
        jnp.broadcast_to(x_t[:, None, :], (TI, N, DIM)),
        jnp.broadcast_to(x_full[None, :, :], (TI, N, DIM)),
        d_t[:, :, None]], axis=-1)
    h2d = _silu(_dotd(e.reshape(TI * N, 2 * DIM + 1), w1_ref[...])
                + b1_ref[...])
    m2d = _silu(_dotd(h2d, w2_ref[...]) + b2_ref[...])
    cw1 = _silu(_dotd(m2d, c1_ref[...]) + bc1_ref[...])
    cw = (_dotd(cw1, c2_ref[...]) + bc2_ref[0, 0]).reshape(TI, N)

    m_i = jnp.sum(m2d.reshape(TI, N, MDIM), axis=1)
    delta_t = jax.lax.dot_general(
        cw, rel, (((1,), (1,)), ((0,), (0,))),
        preferred_element_type=jnp.float32)
    c_out = ci_t + delta_t

    mu = jnp.mean(x_t, axis=1, keepdims=True)
    xc = x_t - mu
    var = jnp.mean(xc * xc, axis=1, keepdims=True)
    normed = xc * jax.lax.rsqrt(var + 1e-5) * lng_ref[...] + lnb_ref[...]
    node_in = jnp.concatenate([normed, m_i], axis=1)
    h2 = _silu(_dot(node_in, n1_ref[...]) + bn1_ref[...])
    x_out = _dot(h2, n2_ref[...]) + bn2_ref[...] + x_t
    return x_out, c_out


def _layer1_kernel(feats_ref, ft_ref, coors_ref, ci_ref, time_ref, emb_ref,
                   t1w_ref, t1b_ref, t2w_ref, t2b_ref, t3w_ref, t3b_ref,
                   *rest):
    (w1_ref, b1_ref, w2_ref, b2_ref,
     c1_ref, bc1_ref, c2_ref, bc2_ref, lng_ref, lnb_ref,
     n1_ref, bn1_ref, n2_ref, bn2_ref, out_x_ref, out_c_ref) = rest

    t = time_ref[0, 0, 0]
    tv = _selu(t * t1w_ref[...] + t1b_ref[...])
    tv = _selu(_dot(tv, t2w_ref[...]) + t2b_ref[...])
    tv = _dot(tv, t3w_ref[...]) + t3b_ref[...]

    f = feats_ref[0]
    tok = jax.lax.broadcasted_iota(jnp.int32, (32, N), 0)
    onehot = (tok == f).astype(jnp.float32)
    emb = emb_ref[...]
    x_full = _dotg(onehot, emb, (((0,), (0,)))) + tv
    f_t = ft_ref[0, 0]
    tok_t = jax.lax.broadcasted_iota(jnp.int32, (32, TI), 0)
    oh_t = (tok_t == f_t).astype(jnp.float32)
    x_t = _dotg(oh_t, emb, (((0,), (0,)))) + tv

    x_out, c_out = _pair_tile(
        x_t, x_full, coors_ref[0], ci_ref[0],
        w1_ref, b1_ref, w2_ref, b2_ref,
        c1_ref, bc1_ref, c2_ref, bc2_ref, lng_ref, lnb_ref,
        n1_ref, bn1_ref, n2_ref, bn2_ref)
    out_x_ref[0] = x_out
    out_c_ref[0] = c_out


def _layer2_kernel(xt_ref, xf_ref, coors_ref, ci_ref, *rest):
    (w1_ref, b1_ref, w2_ref, b2_ref,
     c1_ref, bc1_ref, c2_ref, bc2_ref, lng_ref, lnb_ref,
     n1_ref, bn1_ref, n2_ref, bn2_ref, out_x_ref, out_c_ref) = rest
    x_out, c_out = _pair_tile(
        xt_ref[0], xf_ref[0], coors_ref[0], ci_ref[0],
        w1_ref, b1_ref, w2_ref, b2_ref,
        c1_ref, bc1_ref, c2_ref, bc2_ref, lng_ref, lnb_ref,
        n1_ref, bn1_ref, n2_ref, bn2_ref)
    out_x_ref[0] = x_out
    out_c_ref[0] = c_out


def _layer_params(lp):
    W1, b1 = lp['e1']
    W2, b2 = lp['e2']
    C1, bc1 = lp['c1']
    C2, bc2 = lp['c2']
    n1w, n1b = lp['n1']
    n2w, n2b = lp['n2']
    return (
        W1,
        b1.reshape(1, -1),
        W2,
        b2.reshape(1, -1),
        C1,
        bc1.reshape(1, -1),
        C2,
        bc2.reshape(1, 1),
        lp['ln_g'].reshape(1, -1), lp['ln_b'].reshape(1, -1),
        n1w, n1b.reshape(1, -1), n2w, n2b.reshape(1, -1),
    )


def _bcast(shape):
    nd = len(shape)
    return pl.BlockSpec(shape, lambda b, i: (0,) * nd)


_LAYER_OUT = lambda B: (
    [pl.BlockSpec((1, TI, DIM), lambda b, i: (b, i, 0)),
     pl.BlockSpec((1, TI, 3), lambda b, i: (b, i, 0))],
    [jax.ShapeDtypeStruct((B, N, DIM), jnp.float32),
     jax.ShapeDtypeStruct((B, N, 3), jnp.float32)],
)


def kernel(feats, coors, time, params):
    B = feats.shape[0]
    emb = jnp.zeros((32, DIM), jnp.float32).at[:NTOK].set(params['token_emb'])
    t1w, t1b = params['t1']
    t2w, t2b = params['t2']
    t3w, t3b = params['t3']

    feats3 = feats.astype(jnp.int32).reshape(B, 1, N)
    feats4 = feats.astype(jnp.int32).reshape(B, NI, 1, TI)
    time3 = time.reshape(B, 1, 1)

    lps = [_layer_params(lp) for lp in params['layers']]
    cparams = pltpu.CompilerParams(
        dimension_semantics=("arbitrary", "arbitrary"))

    out_specs, out_shape = _LAYER_OUT(B)

    in_specs1 = [
        pl.BlockSpec((1, 1, N), lambda b, i: (b, 0, 0)),
        pl.BlockSpec((1, 1, 1, TI), lambda b, i: (b, i, 0, 0)),
        pl.BlockSpec((1, N, 3), lambda b, i: (b, 0, 0)),
        pl.BlockSpec((1, TI, 3), lambda b, i: (b, i, 0)),
        pl.BlockSpec((1, 1, 1), lambda b, i: (b, 0, 0)),
        _bcast(emb.shape),
        _bcast((1, TDIM)), _bcast((1, TDIM)),
        _bcast((TDIM, TDIM)), _bcast((1, TDIM)),
        _bcast((TDIM, DIM)), _bcast((1, DIM)),
    ] + [_bcast(a.shape) for a in lps[0]]
    x1, c1 = pl.pallas_call(
        _layer1_kernel,
        grid=(B, NI),
        in_specs=in_specs1,
        out_specs=out_specs,
        out_shape=out_shape,
        compiler_params=cparams,
    )(feats3, feats4, coors, coors, time3, emb,
      t1w.reshape(1, TDIM), t1b.reshape(1, TDIM), t2w, t2b.reshape(1, TDIM),
      t3w, t3b.reshape(1, DIM), *lps[0])

    in_specs2 = [
        pl.BlockSpec((1, TI, DIM), lambda b, i: (b, i, 0)),
        pl.BlockSpec((1, N, DIM), lambda b, i: (b, 0, 0)),
        pl.BlockSpec((1, N, 3), lambda b, i: (b, 0, 0)),
        pl.BlockSpec((1, TI, 3), lambda b, i: (b, i, 0)),
    ] + [_bcast(a.shape) for a in lps[1]]
    x2, c2 = pl.pallas_call(
        _layer2_kernel,
        grid=(B, NI),
        in_specs=in_specs2,
        out_specs=out_specs,
        out_shape=out_shape,
        compiler_params=cparams,
    )(x1, x1, c1, c1, *lps[1])

    return (x2, c2)


# speedup vs baseline: 1.0392x; 1.0392x over previous
import jax
import jax.numpy as jnp
from jax.experimental import pallas as pl
from jax.experimental.pallas import tpu as pltpu

DEPTH = 2
DIM = 64
NTOK = 21
TDIM = 16
MDIM = 16
N = 256
TI = 64
NI = N // TI

_SELU_ALPHA = 1.6732632423543772
_SELU_SCALE = 1.0507009873554805
_HI = jax.lax.Precision.HIGHEST


def _silu(x):
    return x * jax.nn.sigmoid(x)


def _selu(x):
    return _SELU_SCALE * jnp.where(x > 0, x, _SELU_ALPHA * (jnp.exp(x) - 1.0))


def _dot(a, b):
    return jnp.dot(a, b, preferred_element_type=jnp.float32, precision=_HI)


def _dotg(a, b, dims):
    return jax.lax.dot_general(a, b, (dims, ((), ())),
                               preferred_element_type=jnp.float32,
                               precision=_HI)


def _dotd(a, b):
    return jnp.dot(a, b, preferred_element_type=jnp.float32)


def _pair_tile(x_t, x_full, c_full, ci_t,
               w1_ref, b1_ref, w2_ref, b2_ref,
               c1_ref, bc1_ref, c2_ref, bc2_ref, lng_ref, lnb_ref,
               n1_ref, bn1_ref, n2_ref, bn2_ref):
    rel = ci_t[:, None, :] - c_full[None, :, :]
    d_t = jnp.sum(rel * rel, axis=-1)

    e = jnp.concatenate([
        jnp.broadcast_to(x_t[:, None, :], (TI, N, DIM)),
        jnp.broadcast_to(x_full[None, :, :], (TI, N, DIM)),
        d_t[:, :, None]], axis=-1)
    h2d = _silu(_dotd(e.reshape(TI * N, 2 * DIM + 1), w1_ref[...])
                + b1_ref[...])
    m2d = _silu(_dotd(h2d, w2_ref[...]) + b2_ref[...])
    cw1 = _silu(_dotd(m2d, c1_ref[...]) + bc1_ref[...])
    cw = (_dotd(cw1, c2_ref[...]) + bc2_ref[0, 0]).reshape(TI, N)

    m_i = jnp.sum(m2d.reshape(TI, N, MDIM), axis=1)
    delta_t = jax.lax.dot_general(
        cw, rel, (((1,), (1,)), ((0,), (0,))),
        preferred_element_type=jnp.float32)
    c_out = ci_t + delta_t

    mu = jnp.mean(x_t, axis=1, keepdims=True)
    xc = x_t - mu
    var = jnp.mean(xc * xc, axis=1, keepdims=True)
    normed = xc * jax.lax.rsqrt(var + 1e-5) * lng_ref[...] + lnb_ref[...]
    node_in = jnp.concatenate([normed, m_i], axis=1)
    h2 = _silu(_dot(node_in, n1_ref[...]) + bn1_ref[...])
    x_out = _dot(h2, n2_ref[...]) + bn2_ref[...] + x_t
    return x_out, c_out


def _layer1_kernel(feats_ref, ft_ref, coors_ref, ci_ref, time_ref, emb_ref,
                   t1w_ref, t1b_ref, t2w_ref, t2b_ref, t3w_ref, t3b_ref,
                   *rest):
    (w1_ref, b1_ref, w2_ref, b2_ref,
     c1_ref, bc1_ref, c2_ref, bc2_ref, lng_ref, lnb_ref,
     n1_ref, bn1_ref, n2_ref, bn2_ref, out_x_ref, out_c_ref) = rest

    t = time_ref[0, 0, 0]
    tv = _selu(t * t1w_ref[...] + t1b_ref[...])
    tv = _selu(_dot(tv, t2w_ref[...]) + t2b_ref[...])
    tv = _dot(tv, t3w_ref[...]) + t3b_ref[...]

    f = feats_ref[0]
    tok = jax.lax.broadcasted_iota(jnp.int32, (32, N), 0)
    onehot = (tok == f).astype(jnp.float32)
    emb = emb_ref[...]
    x_full = _dotg(onehot, emb, (((0,), (0,)))) + tv
    f_t = ft_ref[0, 0]
    tok_t = jax.lax.broadcasted_iota(jnp.int32, (32, TI), 0)
    oh_t = (tok_t == f_t).astype(jnp.float32)
    x_t = _dotg(oh_t, emb, (((0,), (0,)))) + tv

    x_out, c_out = _pair_tile(
        x_t, x_full, coors_ref[0], ci_ref[0],
        w1_ref, b1_ref, w2_ref, b2_ref,
        c1_ref, bc1_ref, c2_ref, bc2_ref, lng_ref, lnb_ref,
        n1_ref, bn1_ref, n2_ref, bn2_ref)
    out_x_ref[0] = x_out
    out_c_ref[0] = c_out


def _layer2_kernel(xt_ref, xf_ref, coors_ref, ci_ref, *rest):
    (w1_ref, b1_ref, w2_ref, b2_ref,
     c1_ref, bc1_ref, c2_ref, bc2_ref, lng_ref, lnb_ref,
     n1_ref, bn1_ref, n2_ref, bn2_ref, out_x_ref, out_c_ref) = rest
    x_out, c_out = _pair_tile(
        xt_ref[0], xf_ref[0], coors_ref[0], ci_ref[0],
        w1_ref, b1_ref, w2_ref, b2_ref,
        c1_ref, bc1_ref, c2_ref, bc2_ref, lng_ref, lnb_ref,
        n1_ref, bn1_ref, n2_ref, bn2_ref)
    out_x_ref[0] = x_out
    out_c_ref[0] = c_out


def _layer_params(lp):
    W1, b1 = lp['e1']
    W2, b2 = lp['e2']
    C1, bc1 = lp['c1']
    C2, bc2 = lp['c2']
    n1w, n1b = lp['n1']
    n2w, n2b = lp['n2']
    return (
        W1,
        b1.reshape(1, -1),
        W2,
        b2.reshape(1, -1),
        C1,
        bc1.reshape(1, -1),
        C2,
        bc2.reshape(1, 1),
        lp['ln_g'].reshape(1, -1), lp['ln_b'].reshape(1, -1),
        n1w, n1b.reshape(1, -1), n2w, n2b.reshape(1, -1),
    )


def _bcast(shape):
    nd = len(shape)
    return pl.BlockSpec(shape, lambda b, i: (0,) * nd)


_LAYER_OUT = lambda B: (
    [pl.BlockSpec((1, TI, DIM), lambda b, i: (b, i, 0)),
     pl.BlockSpec((1, TI, 3), lambda b, i: (b, i, 0))],
    [jax.ShapeDtypeStruct((B, N, DIM), jnp.float32),
     jax.ShapeDtypeStruct((B, N, 3), jnp.float32)],
)


def kernel(feats, coors, time, params):
    B = feats.shape[0]
    emb = jnp.zeros((32, DIM), jnp.float32).at[:NTOK].set(params['token_emb'])
    t1w, t1b = params['t1']
    t2w, t2b = params['t2']
    t3w, t3b = params['t3']

    feats3 = feats.astype(jnp.int32).reshape(B, 1, N)
    feats4 = feats.astype(jnp.int32).reshape(B, NI, 1, TI)
    time3 = time.reshape(B, 1, 1)

    lps = [_layer_params(lp) for lp in params['layers']]
    cparams = pltpu.CompilerParams(
        dimension_semantics=("arbitrary", "arbitrary"))

    out_specs, out_shape = _LAYER_OUT(B)

    in_specs1 = [
        pl.BlockSpec((1, 1, N), lambda b, i: (b, 0, 0)),
        pl.BlockSpec((1, 1, 1, TI), lambda b, i: (b, i, 0, 0)),
        pl.BlockSpec((1, N, 3), lambda b, i: (b, 0, 0)),
        pl.BlockSpec((1, TI, 3), lambda b, i: (b, i, 0)),
        pl.BlockSpec((1, 1, 1), lambda b, i: (b, 0, 0)),
        _bcast(emb.shape),
        _bcast((1, TDIM)), _bcast((1, TDIM)),
        _bcast((TDIM, TDIM)), _bcast((1, TDIM)),
        _bcast((TDIM, DIM)), _bcast((1, DIM)),
    ] + [_bcast(a.shape) for a in lps[0]]
    x1, c1 = pl.pallas_call(
        _layer1_kernel,
        grid=(B, NI),
        in_specs=in_specs1,
        out_specs=out_specs,
        out_shape=out_shape,
        compiler_params=cparams,
    )(feats3, feats4, coors, coors, time3, emb,
      t1w.reshape(1, TDIM), t1b.reshape(1, TDIM), t2w, t2b.reshape(1, TDIM),
      t3w, t3b.reshape(1, DIM), *lps[0])

    in_specs2 = [
        pl.BlockSpec((1, TI, DIM), lambda b, i: (b, i, 0)),
        pl.BlockSpec((1, N, DIM), lambda b, i: (b, 0, 0)),
        pl.BlockSpec((1, N, 3), lambda b, i: (b, 0, 0)),
        pl.BlockSpec((1, TI, 3), lambda b, i: (b, i, 0)),
    ] + [_bcast(a.shape) for a in lps[1]]
    x2, c2 = pl.pallas_call(
        _layer2_kernel,
        grid=(B, NI),
        in_specs=in_specs2,
        out_specs=out_specs,
        out_shape=out_shape,
        compiler_params=cparams,
    )(x1, x1, c1, c1, *lps[1])

    return (x2, c2)
